# Initial kernel scaffold; baseline (speedup 1.0000x reference)
import jax
import jax.numpy as jnp
from jax.experimental import pallas as pl


def kernel(src_points, src_feats, tgt_points, tgt_feats, params):
    raise NotImplementedError("write your pallas kernel here")


# speedup vs baseline: 5.9805x; 5.9805x over previous
import functools

import jax
import jax.numpy as jnp
from jax import lax
from jax.experimental import pallas as pl
from jax.experimental.pallas import tpu as pltpu
from jax.experimental.pallas import tpu_sc as plsc

F32 = jnp.float32
N = 2048
NS = 4096
KNN = 10
NCORES = 2
NSUB = 16
NW = NCORES * NSUB
QPW = NS // NW
QC = 8
IPC = QC * KNN
NCHUNK = QPW // QC


def _leaky(x):
    return jnp.where(x >= 0, x, 0.2 * x)


_HI = lax.Precision.HIGHEST


def _dot(a, b, precision=None):
    return jnp.dot(a, b, preferred_element_type=F32, precision=precision)


def _hdot(a, b):
    return jnp.dot(a, b, preferred_element_type=F32, precision=_HI)


def _cdot(a, b, precision=None):
    return lax.dot_general(a, b, (((1,), (1,)), ((), ())),
                           preferred_element_type=F32, precision=precision)


def _tdot(a, b):
    return lax.dot_general(a, b, (((0,), (0,)), ((), ())),
                           preferred_element_type=F32)


def _knn_body(pq_ref, pa_ref, idx_ref):
    b = pl.program_id(0)
    pq = pq_ref[...]
    pa = pa_ref[...]
    g = _cdot(pq, pa)
    sqq = jnp.sum(pq * pq, axis=1, keepdims=True)
    sqa = jnp.transpose(jnp.sum(pa * pa, axis=1, keepdims=True))
    d = ((-2.0 * g) + sqq) + sqa
    d = jnp.maximum(d, 1e-12)
    colid = lax.broadcasted_iota(jnp.int32, (256, 2048), 1)
    big = jnp.float32(3.0e38)
    outs = []
    for _ in range(KNN + 1):
        rmin = jnp.min(d, axis=1, keepdims=True)
        amin = jnp.min(jnp.where(d <= rmin, colid, 2048), axis=1, keepdims=True)
        outs.append(amin)
        d = jnp.where(colid == amin, big, d)
    idx_ref[...] = jnp.concatenate(outs[1:], axis=1) + (b // 8) * 2048


def _knn(pts8):
    return pl.pallas_call(
        _knn_body,
        grid=(16,),
        in_specs=[
            pl.BlockSpec((256, 8), lambda b: (b, 0)),
            pl.BlockSpec((2048, 8), lambda b: (b // 8, 0)),
        ],
        out_specs=pl.BlockSpec((256, KNN), lambda b: (b, 0)),
        out_shape=jax.ShapeDtypeStruct((NS, KNN), jnp.int32),
    )(pts8, pts8)


@functools.lru_cache(maxsize=None)
def _make_gather_reduce(C):
    mesh = plsc.VectorSubcoreMesh(core_axis_name="c", subcore_axis_name="s")

    @functools.partial(
        pl.kernel,
        mesh=mesh,
        out_type=[jax.ShapeDtypeStruct((NS, C), F32)] * 3,
        scratch_types=[
            pltpu.VMEM((QPW * KNN,), jnp.int32),
            pltpu.VMEM((IPC, C), F32),
            pltpu.VMEM((IPC, C), F32),
            pltpu.VMEM((QC, C), F32),
            pltpu.VMEM((QC, C), F32),
            pltpu.VMEM((QC, C), F32),
            pltpu.SemaphoreType.DMA,
            pltpu.SemaphoreType.DMA,
        ],
    )
    def gr(table_hbm, idx_hbm, s_hbm, q_hbm, m_hbm,
           idx_v, rows_a, rows_b, s_v, q_v, m_v, sem_a, sem_b):
        wid = lax.axis_index("s") * NCORES + lax.axis_index("c")
        qbase = wid * QPW
        pltpu.sync_copy(idx_hbm.at[pl.ds(qbase * KNN, QPW * KNN)], idx_v)

        def start(c, rows, sem):
            pltpu.async_copy(table_hbm.at[idx_v.at[pl.ds(c * IPC, IPC)]],
                             rows, sem)

        def wait(rows, sem):
            pltpu.make_async_copy(
                table_hbm.at[idx_v.at[pl.ds(0, IPC)]], rows, sem).wait()

        def compute(c, rows):
            def qloop(qi, carry):
                for j in range(C // 16):
                    sl = pl.ds(j * 16, 16)
                    r = rows[qi * KNN, sl]
                    acc_s = r
                    acc_q = r * r
                    acc_m = r
                    for k in range(1, KNN):
                        r = rows[qi * KNN + k, sl]
                        acc_s = acc_s + r
                        acc_q = acc_q + r * r
                        acc_m = jnp.maximum(acc_m, r)
                    s_v[qi, sl] = acc_s
                    q_v[qi, sl] = acc_q
                    m_v[qi, sl] = acc_m
                return carry
            lax.fori_loop(0, QC, qloop, 0)
            off = qbase + c * QC
            pltpu.sync_copy(s_v, s_hbm.at[pl.ds(off, QC)])
            pltpu.sync_copy(q_v, q_hbm.at[pl.ds(off, QC)])
            pltpu.sync_copy(m_v, m_hbm.at[pl.ds(off, QC)])

        start(0, rows_a, sem_a)

        def body(g2, carry):
            c0 = g2 * 2
            start(c0 + 1, rows_b, sem_b)
            wait(rows_a, sem_a)
            compute(c0, rows_a)

            @pl.when(g2 < NCHUNK // 2 - 1)
            def _():
                start(c0 + 2, rows_a, sem_a)

            wait(rows_b, sem_b)
            compute(c0 + 1, rows_b)
            return carry

        lax.fori_loop(0, NCHUNK // 2, body, 0)

    return gr


def _gather_reduce(table, idx_flat):
    return _make_gather_reduce(table.shape[1])(table, idx_flat)


def _lin1_body(x_ref, w_ref, a_ref, b_ref):
    y = _dot(x_ref[...], w_ref[...])
    a_ref[...] = y[:, :256]
    b_ref[...] = y[:, 256:]


def _lin1(x, wt):
    return pl.pallas_call(
        _lin1_body,
        out_shape=[jax.ShapeDtypeStruct((NS, 256), F32)] * 2,
    )(x, wt)


def _edge_stats(ap, b, s, q, mx):
    a = ap - b
    nk = jnp.float32(N * KNN)
    kf = jnp.float32(KNN)
    mean = (jnp.sum(a, axis=0, keepdims=True) * kf
            + jnp.sum(s, axis=0, keepdims=True)) / nk
    msq = (jnp.sum(a * a, axis=0, keepdims=True) * kf
           + 2.0 * jnp.sum(a * s, axis=0, keepdims=True)
           + jnp.sum(q, axis=0, keepdims=True)) / nk
    var = msq - mean * mean
    r = lax.rsqrt(var + 1e-5)
    return _leaky((a + mx - mean) * r)


def _stage2_body(a_ref, b_ref, s_ref, q_ref, m_ref, w_ref,
                 x1_ref, a2_ref, b2_ref):
    x1 = _edge_stats(a_ref[...], b_ref[...], s_ref[...], q_ref[...],
                     m_ref[...])
    x1_ref[...] = x1
    y = _dot(x1, w_ref[...])
    a2_ref[...] = y[:, :512]
    b2_ref[...] = y[:, 512:]


def _stage2(a1, b1, s1, q1, m1, wt2):
    spec = pl.BlockSpec((N, 256), lambda h: (h, 0))
    return pl.pallas_call(
        _stage2_body,
        grid=(2,),
        in_specs=[spec, spec, spec, spec, spec,
                  pl.BlockSpec((256, 1024), lambda h: (0, 0))],
        out_specs=[pl.BlockSpec((N, 256), lambda h: (h, 0)),
                   pl.BlockSpec((N, 512), lambda h: (h, 0)),
                   pl.BlockSpec((N, 512), lambda h: (h, 0))],
        out_shape=[jax.ShapeDtypeStruct((NS, 256), F32),
                   jax.ShapeDtypeStruct((NS, 512), F32),
                   jax.ShapeDtypeStruct((NS, 512), F32)],
    )(a1, b1, s1, q1, m1, wt2)


def _stage3a_body(a_ref, b_ref, s_ref, q_ref, m_ref, x2_ref):
    x2_ref[...] = _edge_stats(a_ref[...], b_ref[...], s_ref[...], q_ref[...],
                              m_ref[...])


def _stage3b_body(x0_ref, x1_ref, x2_ref, wa_ref, wb_ref, wc_ref, x3_ref):
    y = (_dot(x0_ref[...], wa_ref[...])
         + _dot(x1_ref[...], wb_ref[...])
         + _dot(x2_ref[...], wc_ref[...]))
    mu = jnp.mean(y, axis=0, keepdims=True)
    d = y - mu
    v = jnp.mean(d * d, axis=0, keepdims=True)
    x3_ref[...] = _leaky(d * lax.rsqrt(v + 1e-5))


def _stage3(x0, x1, a2, b2, s2, q2, m2, wa, wb, wc):
    spec256 = pl.BlockSpec((N, 256), lambda h: (h, 0))
    spec512 = pl.BlockSpec((N, 512), lambda h: (h, 0))
    wspec = lambda shape: pl.BlockSpec(shape, lambda h: (0, 0))
    x2 = pl.pallas_call(
        _stage3a_body,
        grid=(2,),
        in_specs=[spec512, spec512, spec512, spec512, spec512],
        out_specs=spec512,
        out_shape=jax.ShapeDtypeStruct((NS, 512), F32),
    )(a2, b2, s2, q2, m2)
    return pl.pallas_call(
        _stage3b_body,
        grid=(2,),
        in_specs=[spec256, spec256, spec512,
                  wspec((256, 256)), wspec((256, 256)), wspec((512, 256))],
        out_specs=spec256,
        out_shape=jax.ShapeDtypeStruct((NS, 256), F32),
    )(x0, x1, x2, wa, wb, wc)


def _linear_body(x_ref, w_ref, b_ref, o_ref):
    o_ref[...] = (_dot(x_ref[...], w_ref[...])
                  + b_ref[...])


def _linear(x, wt, b):
    m, cout = x.shape[0], wt.shape[1]
    return pl.pallas_call(
        _linear_body,
        out_shape=jax.ShapeDtypeStruct((m, cout), F32),
    )(x, wt, b)


def _mha_body(q_ref, k_ref, v_ref, o_ref):
    s = _cdot(q_ref[0], k_ref[0]) * 0.125
    m = jnp.max(s, axis=1, keepdims=True)
    e = jnp.exp(s - m)
    den = jnp.sum(e, axis=1, keepdims=True)
    o_ref[0] = _dot(e, v_ref[0]) / den


def _mha(qp, kvp):
    q3 = qp.reshape(N, 4, 64).transpose(1, 0, 2)
    k3 = kvp[:, :256].reshape(N, 4, 64).transpose(1, 0, 2)
    v3 = kvp[:, 256:].reshape(N, 4, 64).transpose(1, 0, 2)
    hspec = pl.BlockSpec((1, N, 64), lambda h: (h, 0, 0))
    o3 = pl.pallas_call(
        _mha_body,
        grid=(4,),
        in_specs=[hspec, hspec, hspec],
        out_specs=hspec,
        out_shape=jax.ShapeDtypeStruct((4, N, 64), F32),
    )(q3, k3, v3)
    return o3.transpose(1, 0, 2).reshape(N, 256)


def _mlp1_body(s_ref, a_ref, wa_ref, wb_ref, wm_ref, mb_ref, b_ref, o_ref):
    msg = _dot(a_ref[...], wm_ref[...]) + mb_ref[...]
    y = (_dot(s_ref[...], wa_ref[...])
         + _dot(msg, wb_ref[...])
         + b_ref[...])
    mu = jnp.mean(y, axis=0, keepdims=True)
    d = y - mu
    v = jnp.mean(d * d, axis=0, keepdims=True)
    o_ref[...] = jnp.maximum(d * lax.rsqrt(v + 1e-5), 0.0)


def _mlp1(s, a, wa, wb, wm, mb, b):
    return pl.pallas_call(
        _mlp1_body,
        out_shape=jax.ShapeDtypeStruct((N, 512), F32),
    )(s, a, wa, wb, wm, mb, b)


def _mlp2_body(h_ref, w_ref, b_ref, res_ref, o_ref):
    o_ref[...] = (_dot(h_ref[...], w_ref[...])
                  + b_ref[...] + res_ref[...])


def _mlp2(h, wt, b, res):
    return pl.pallas_call(
        _mlp2_body,
        out_shape=jax.ShapeDtypeStruct((N, 256), F32),
    )(h, wt, b, res)


def _proj_body(x_ref, wg_ref, bg_ref, ws_ref, bs_ref, fc_ref, sc_ref):
    fc = (jnp.dot(x_ref[...], wg_ref[...], preferred_element_type=F32)
          + bg_ref[...])
    fc_ref[...] = fc
    sc_ref[...] = (_dot(fc, ws_ref[...])
                   + bs_ref[...])


def _proj(x, wg, bg, ws, bs):
    return pl.pallas_call(
        _proj_body,
        out_shape=[jax.ShapeDtypeStruct((NS, 256), F32),
                   jax.ShapeDtypeStruct((NS, 1), F32)],
    )(x, wg, bg, ws, bs)


def _norm_inner(fc_ref, eps_ref):
    fg = fc_ref[...]
    nr = jnp.maximum(jnp.sqrt(jnp.sum(fg * fg, axis=1, keepdims=True)), 1e-12)
    g = fg / nr
    temp = jnp.exp(eps_ref[0, 0]) + 0.03
    return _cdot(g[:N], g[N:]) / temp


def _final1_body(fc_ref, ts_ref, eps_ref, s1_ref):
    inner = _norm_inner(fc_ref, eps_ref)
    m1 = jnp.max(inner, axis=1, keepdims=True)
    e1 = jnp.exp(inner - m1)
    s1_ref[...] = (_dot(e1, ts_ref[...])
                   / jnp.sum(e1, axis=1, keepdims=True))


def _final2_body(fc_ref, ss_ref, eps_ref, s2_ref):
    inner = _norm_inner(fc_ref, eps_ref)
    m2 = jnp.max(inner, axis=0, keepdims=True)
    e2 = jnp.exp(inner - m2)
    p2 = e2 / jnp.sum(e2, axis=0, keepdims=True)
    s2_ref[...] = _tdot(p2, ss_ref[...])


def _final(fc, ss, ts, eps):
    s1 = pl.pallas_call(
        _final1_body,
        out_shape=jax.ShapeDtypeStruct((N, 1), F32),
    )(fc, ts, eps)
    s2 = pl.pallas_call(
        _final2_body,
        out_shape=jax.ShapeDtypeStruct((N, 1), F32),
    )(fc, ss, eps)
    return s1, s2


def _prep_sa(p):
    c1 = p['conv1']
    wt1 = jnp.concatenate([c1[:, :256].T, c1[:, 256:].T], axis=1)
    c2 = p['conv2']
    wt2 = jnp.concatenate([c2[:, :256].T, c2[:, 256:].T], axis=1)
    c3 = p['conv3']
    return wt1, wt2, c3[:, :256].T, c3[:, 256:512].T, c3[:, 512:].T


def _prep_ap(p):
    perm = jnp.arange(256).reshape(64, 4).T.reshape(-1)
    qt = p['q_w'][perm, :].T
    qb = p['q_b'][perm][None, :]
    kvt = jnp.concatenate([p['k_w'][perm, :].T, p['v_w'][perm, :].T], axis=1)
    kvb = jnp.concatenate([p['k_b'][perm], p['v_b'][perm]])[None, :]
    wm = p['m_w'][:, perm].T
    mb = p['m_b'][None, :]
    wa1 = p['mlp1_w'][:, :256].T
    wb1 = p['mlp1_w'][:, 256:].T
    b1 = p['mlp1_b'][None, :]
    w2t = p['mlp2_w'].T
    b2 = p['mlp2_b'][None, :]
    return qt, qb, kvt, kvb, wa1, wb1, wm, mb, b1, w2t, b2


def _self_att(X, idx_flat, wt1, wt2, wa, wb, wc):
    a1, b1 = _lin1(X, wt1)
    s1, q1, m1 = _gather_reduce(b1, idx_flat)
    x1, a2, b2 = _stage2(a1, b1, s1, q1, m1, wt2)
    s2, q2, m2 = _gather_reduce(b2, idx_flat)
    return _stage3(X, x1, a2, b2, s2, q2, m2, wa, wb, wc)


def _att_prop(S, T, ap):
    qt, qb, kvt, kvb, wa1, wb1, wm, mb, b1, w2t, b2 = ap
    qp = _linear(S, qt, qb)
    kvp = _linear(T, kvt, kvb)
    ao = _mha(qp, kvp)
    h1 = _mlp1(S, ao, wa1, wb1, wm, mb, b1)
    return _mlp2(h1, w2t, b2, S)


@jax.jit
def _impl(src_points, src_feats, tgt_points, tgt_feats, params):
    X0 = jnp.concatenate([src_feats[0].T, tgt_feats[0].T], axis=0)
    pts = jnp.concatenate([src_points[0].T, tgt_points[0].T], axis=0)
    pts8 = jnp.pad(pts, ((0, 0), (0, 5)))
    idx_flat = _knn(pts8).reshape(-1)

    sa1 = _prep_sa(params['sa1'])
    sa2 = _prep_sa(params['sa2'])
    ap = _prep_ap(params['ap'])

    Xa = _self_att(X0, idx_flat, *sa1)
    sf, tf = Xa[:N], Xa[N:]
    sf = _att_prop(sf, tf, ap)
    tf = _att_prop(tf, sf, ap)
    Xb = jnp.concatenate([sf, tf], axis=0)
    Xc = _self_att(Xb, idx_flat, *sa2)

    fc, sc = _proj(Xc, params['proj_gnn_w'].T, params['proj_gnn_b'][None, :],
                   params['proj_score_w'].T, params['proj_score_b'][None, :])
    ss, ts = sc[:N], sc[N:]
    eps = jnp.reshape(params['epsilon'], (1, 1))
    s1, s2 = _final(fc, ss, ts, eps)
    sal = jnp.concatenate([s1, s2], axis=0)
    return jnp.concatenate([sc, sal, fc], axis=1)


def kernel(src_points, src_feats, tgt_points, tgt_feats, params):
    return _impl(src_points, src_feats, tgt_points, tgt_feats, params)


# speedup vs baseline: 6.0538x; 1.0123x over previous
import functools

import jax
import jax.numpy as jnp
from jax import lax
from jax.experimental import pallas as pl
from jax.experimental.pallas import tpu as pltpu
from jax.experimental.pallas import tpu_sc as plsc

F32 = jnp.float32
N = 2048
NS = 4096
KNN = 10
NCORES = 2
NSUB = 16
NW = NCORES * NSUB
QPW = NS // NW
QC = 8
IPC = QC * KNN
NCHUNK = QPW // QC


def _leaky(x):
    return jnp.where(x >= 0, x, 0.2 * x)


_HI = lax.Precision.HIGHEST


def _dot(a, b, precision=None):
    return jnp.dot(a, b, preferred_element_type=F32, precision=precision)


def _hdot(a, b):
    return jnp.dot(a, b, preferred_element_type=F32, precision=_HI)


def _cdot(a, b, precision=None):
    return lax.dot_general(a, b, (((1,), (1,)), ((), ())),
                           preferred_element_type=F32, precision=precision)


def _tdot(a, b):
    return lax.dot_general(a, b, (((0,), (0,)), ((), ())),
                           preferred_element_type=F32)


def _knn_body(pq_ref, pa_ref, idx_ref, cnt_ref):
    b = pl.program_id(0)
    pq = pq_ref[...]
    pa = pa_ref[...]
    g = _cdot(pq, pa)
    sqq = jnp.sum(pq * pq, axis=1, keepdims=True)
    sqa = jnp.transpose(jnp.sum(pa * pa, axis=1, keepdims=True))
    d = ((-2.0 * g) + sqq) + sqa
    d = jnp.maximum(d, 1e-12)
    colid = lax.broadcasted_iota(jnp.int32, (256, 2048), 1)
    big = jnp.float32(3.0e38)
    outs = []
    cnt = jnp.zeros((1, 2048), F32)
    for j in range(KNN + 1):
        rmin = jnp.min(d, axis=1, keepdims=True)
        amin = jnp.min(jnp.where(d <= rmin, colid, 2048), axis=1, keepdims=True)
        outs.append(amin)
        onehot = colid == amin
        if j > 0:
            cnt = cnt + jnp.sum(onehot.astype(F32), axis=0, keepdims=True)
        d = jnp.where(onehot, big, d)
    idx_ref[...] = jnp.concatenate(outs[1:], axis=1) + (b // 8) * 2048

    @pl.when(b % 8 == 0)
    def _():
        cnt_ref[...] = jnp.zeros((1, 2048), F32)
    cnt_ref[...] += cnt


def _knn(pts8):
    return pl.pallas_call(
        _knn_body,
        grid=(16,),
        in_specs=[
            pl.BlockSpec((256, 8), lambda b: (b, 0)),
            pl.BlockSpec((2048, 8), lambda b: (b // 8, 0)),
        ],
        out_specs=[pl.BlockSpec((256, KNN), lambda b: (b, 0)),
                   pl.BlockSpec((1, 2048), lambda b: (0, b // 8))],
        out_shape=[jax.ShapeDtypeStruct((NS, KNN), jnp.int32),
                   jax.ShapeDtypeStruct((1, NS), F32)],
    )(pts8, pts8)


@functools.lru_cache(maxsize=None)
def _make_gather_reduce(C):
    mesh = plsc.VectorSubcoreMesh(core_axis_name="c", subcore_axis_name="s")

    @functools.partial(
        pl.kernel,
        mesh=mesh,
        out_type=[jax.ShapeDtypeStruct((NS, C), F32)] * 2,
        scratch_types=[
            pltpu.VMEM((QPW * KNN,), jnp.int32),
            pltpu.VMEM((IPC, C), F32),
            pltpu.VMEM((IPC, C), F32),
            pltpu.VMEM((QC, C), F32),
            pltpu.VMEM((QC, C), F32),
            pltpu.SemaphoreType.DMA,
            pltpu.SemaphoreType.DMA,
        ],
    )
    def gr(table_hbm, idx_hbm, s_hbm, m_hbm,
           idx_v, rows_a, rows_b, s_v, m_v, sem_a, sem_b):
        wid = lax.axis_index("s") * NCORES + lax.axis_index("c")
        qbase = wid * QPW
        pltpu.sync_copy(idx_hbm.at[pl.ds(qbase * KNN, QPW * KNN)], idx_v)

        def start(c, rows, sem):
            pltpu.async_copy(table_hbm.at[idx_v.at[pl.ds(c * IPC, IPC)]],
                             rows, sem)

        def wait(rows, sem):
            pltpu.make_async_copy(
                table_hbm.at[idx_v.at[pl.ds(0, IPC)]], rows, sem).wait()

        def compute(c, rows):
            def qloop(qi, carry):
                for j in range(C // 16):
                    sl = pl.ds(j * 16, 16)
                    r = rows[qi * KNN, sl]
                    acc_s = r
                    acc_m = r
                    for k in range(1, KNN):
                        r = rows[qi * KNN + k, sl]
                        acc_s = acc_s + r
                        acc_m = jnp.maximum(acc_m, r)
                    s_v[qi, sl] = acc_s
                    m_v[qi, sl] = acc_m
                return carry
            lax.fori_loop(0, QC, qloop, 0)
            off = qbase + c * QC
            pltpu.sync_copy(s_v, s_hbm.at[pl.ds(off, QC)])
            pltpu.sync_copy(m_v, m_hbm.at[pl.ds(off, QC)])

        start(0, rows_a, sem_a)

        def body(g2, carry):
            c0 = g2 * 2
            start(c0 + 1, rows_b, sem_b)
            wait(rows_a, sem_a)
            compute(c0, rows_a)

            @pl.when(g2 < NCHUNK // 2 - 1)
            def _():
                start(c0 + 2, rows_a, sem_a)

            wait(rows_b, sem_b)
            compute(c0 + 1, rows_b)
            return carry

        lax.fori_loop(0, NCHUNK // 2, body, 0)

    return gr


def _gather_reduce(table, idx_flat):
    return _make_gather_reduce(table.shape[1])(table, idx_flat)


def _lin1_body(x_ref, w_ref, a_ref, b_ref):
    y = _dot(x_ref[...], w_ref[...])
    a_ref[...] = y[:, :256]
    b_ref[...] = y[:, 256:]


def _lin1(x, wt):
    return pl.pallas_call(
        _lin1_body,
        out_shape=[jax.ShapeDtypeStruct((NS, 256), F32)] * 2,
    )(x, wt)


def _edge_stats(ap, b, s, mx, cnt):
    a = ap - b
    nk = jnp.float32(N * KNN)
    kf = jnp.float32(KNN)
    cw = jnp.transpose(cnt)
    mean = (jnp.sum(a, axis=0, keepdims=True) * kf
            + jnp.sum(s, axis=0, keepdims=True)) / nk
    msq = (jnp.sum(a * a, axis=0, keepdims=True) * kf
           + 2.0 * jnp.sum(a * s, axis=0, keepdims=True)
           + jnp.sum(b * b * cw, axis=0, keepdims=True)) / nk
    var = msq - mean * mean
    r = lax.rsqrt(var + 1e-5)
    return _leaky((a + mx - mean) * r)


def _stage2_body(a_ref, b_ref, s_ref, m_ref, cnt_ref, w_ref,
                 x1_ref, a2_ref, b2_ref):
    x1 = _edge_stats(a_ref[...], b_ref[...], s_ref[...], m_ref[...],
                     cnt_ref[...])
    x1_ref[...] = x1
    y = _dot(x1, w_ref[...])
    a2_ref[...] = y[:, :512]
    b2_ref[...] = y[:, 512:]


def _stage2(a1, b1, s1, m1, cnt, wt2):
    spec = pl.BlockSpec((N, 256), lambda h: (h, 0))
    return pl.pallas_call(
        _stage2_body,
        grid=(2,),
        in_specs=[spec, spec, spec, spec,
                  pl.BlockSpec((1, N), lambda h: (0, h)),
                  pl.BlockSpec((256, 1024), lambda h: (0, 0))],
        out_specs=[pl.BlockSpec((N, 256), lambda h: (h, 0)),
                   pl.BlockSpec((N, 512), lambda h: (h, 0)),
                   pl.BlockSpec((N, 512), lambda h: (h, 0))],
        out_shape=[jax.ShapeDtypeStruct((NS, 256), F32),
                   jax.ShapeDtypeStruct((NS, 512), F32),
                   jax.ShapeDtypeStruct((NS, 512), F32)],
    )(a1, b1, s1, m1, cnt, wt2)


def _stage3a_body(a_ref, b_ref, s_ref, m_ref, cnt_ref, x2_ref):
    x2_ref[...] = _edge_stats(a_ref[...], b_ref[...], s_ref[...], m_ref[...],
                              cnt_ref[...])


def _stage3b_body(x0_ref, x1_ref, x2_ref, wa_ref, wb_ref, wc_ref, x3_ref):
    y = (_dot(x0_ref[...], wa_ref[...])
         + _dot(x1_ref[...], wb_ref[...])
         + _dot(x2_ref[...], wc_ref[...]))
    mu = jnp.mean(y, axis=0, keepdims=True)
    d = y - mu
    v = jnp.mean(d * d, axis=0, keepdims=True)
    x3_ref[...] = _leaky(d * lax.rsqrt(v + 1e-5))


def _stage3(x0, x1, a2, b2, s2, m2, cnt, wa, wb, wc):
    spec256 = pl.BlockSpec((N, 256), lambda h: (h, 0))
    spec512 = pl.BlockSpec((N, 512), lambda h: (h, 0))
    wspec = lambda shape: pl.BlockSpec(shape, lambda h: (0, 0))
    x2 = pl.pallas_call(
        _stage3a_body,
        grid=(2,),
        in_specs=[spec512, spec512, spec512, spec512,
                  pl.BlockSpec((1, N), lambda h: (0, h))],
        out_specs=spec512,
        out_shape=jax.ShapeDtypeStruct((NS, 512), F32),
    )(a2, b2, s2, m2, cnt)
    return pl.pallas_call(
        _stage3b_body,
        grid=(2,),
        in_specs=[spec256, spec256, spec512,
                  wspec((256, 256)), wspec((256, 256)), wspec((512, 256))],
        out_specs=spec256,
        out_shape=jax.ShapeDtypeStruct((NS, 256), F32),
    )(x0, x1, x2, wa, wb, wc)


def _linear_body(x_ref, w_ref, b_ref, o_ref):
    o_ref[...] = (_dot(x_ref[...], w_ref[...])
                  + b_ref[...])


def _linear(x, wt, b):
    m, cout = x.shape[0], wt.shape[1]
    return pl.pallas_call(
        _linear_body,
        out_shape=jax.ShapeDtypeStruct((m, cout), F32),
    )(x, wt, b)


def _mha_body(q_ref, k_ref, v_ref, o_ref):
    s = _cdot(q_ref[0], k_ref[0]) * 0.125
    m = jnp.max(s, axis=1, keepdims=True)
    e = jnp.exp(s - m)
    den = jnp.sum(e, axis=1, keepdims=True)
    o_ref[0] = _dot(e, v_ref[0]) / den


def _mha(qp, kvp):
    q3 = qp.reshape(N, 4, 64).transpose(1, 0, 2)
    k3 = kvp[:, :256].reshape(N, 4, 64).transpose(1, 0, 2)
    v3 = kvp[:, 256:].reshape(N, 4, 64).transpose(1, 0, 2)
    hspec = pl.BlockSpec((1, N, 64), lambda h: (h, 0, 0))
    o3 = pl.pallas_call(
        _mha_body,
        grid=(4,),
        in_specs=[hspec, hspec, hspec],
        out_specs=hspec,
        out_shape=jax.ShapeDtypeStruct((4, N, 64), F32),
    )(q3, k3, v3)
    return o3.transpose(1, 0, 2).reshape(N, 256)


def _mlp1_body(s_ref, a_ref, wa_ref, wb_ref, wm_ref, mb_ref, b_ref, o_ref):
    msg = _dot(a_ref[...], wm_ref[...]) + mb_ref[...]
    y = (_dot(s_ref[...], wa_ref[...])
         + _dot(msg, wb_ref[...])
         + b_ref[...])
    mu = jnp.mean(y, axis=0, keepdims=True)
    d = y - mu
    v = jnp.mean(d * d, axis=0, keepdims=True)
    o_ref[...] = jnp.maximum(d * lax.rsqrt(v + 1e-5), 0.0)


def _mlp1(s, a, wa, wb, wm, mb, b):
    return pl.pallas_call(
        _mlp1_body,
        out_shape=jax.ShapeDtypeStruct((N, 512), F32),
    )(s, a, wa, wb, wm, mb, b)


def _mlp2_body(h_ref, w_ref, b_ref, res_ref, o_ref):
    o_ref[...] = (_dot(h_ref[...], w_ref[...])
                  + b_ref[...] + res_ref[...])


def _mlp2(h, wt, b, res):
    return pl.pallas_call(
        _mlp2_body,
        out_shape=jax.ShapeDtypeStruct((N, 256), F32),
    )(h, wt, b, res)


def _proj_body(x_ref, wg_ref, bg_ref, ws_ref, bs_ref, fc_ref, sc_ref):
    fc = (jnp.dot(x_ref[...], wg_ref[...], preferred_element_type=F32)
          + bg_ref[...])
    fc_ref[...] = fc
    sc_ref[...] = (_dot(fc, ws_ref[...])
                   + bs_ref[...])


def _proj(x, wg, bg, ws, bs):
    return pl.pallas_call(
        _proj_body,
        out_shape=[jax.ShapeDtypeStruct((NS, 256), F32),
                   jax.ShapeDtypeStruct((NS, 1), F32)],
    )(x, wg, bg, ws, bs)


def _norm_inner(fc_ref, eps_ref):
    fg = fc_ref[...]
    nr = jnp.maximum(jnp.sqrt(jnp.sum(fg * fg, axis=1, keepdims=True)), 1e-12)
    g = fg / nr
    temp = jnp.exp(eps_ref[0, 0]) + 0.03
    return _cdot(g[:N], g[N:]) / temp


def _final1_body(fc_ref, ts_ref, eps_ref, s1_ref):
    inner = _norm_inner(fc_ref, eps_ref)
    m1 = jnp.max(inner, axis=1, keepdims=True)
    e1 = jnp.exp(inner - m1)
    s1_ref[...] = (_dot(e1, ts_ref[...])
                   / jnp.sum(e1, axis=1, keepdims=True))


def _final2_body(fc_ref, ss_ref, eps_ref, s2_ref):
    inner = _norm_inner(fc_ref, eps_ref)
    m2 = jnp.max(inner, axis=0, keepdims=True)
    e2 = jnp.exp(inner - m2)
    p2 = e2 / jnp.sum(e2, axis=0, keepdims=True)
    s2_ref[...] = _tdot(p2, ss_ref[...])


def _final(fc, ss, ts, eps):
    s1 = pl.pallas_call(
        _final1_body,
        out_shape=jax.ShapeDtypeStruct((N, 1), F32),
    )(fc, ts, eps)
    s2 = pl.pallas_call(
        _final2_body,
        out_shape=jax.ShapeDtypeStruct((N, 1), F32),
    )(fc, ss, eps)
    return s1, s2


def _prep_sa(p):
    c1 = p['conv1']
    wt1 = jnp.concatenate([c1[:, :256].T, c1[:, 256:].T], axis=1)
    c2 = p['conv2']
    wt2 = jnp.concatenate([c2[:, :256].T, c2[:, 256:].T], axis=1)
    c3 = p['conv3']
    return wt1, wt2, c3[:, :256].T, c3[:, 256:512].T, c3[:, 512:].T


def _prep_ap(p):
    perm = jnp.arange(256).reshape(64, 4).T.reshape(-1)
    qt = p['q_w'][perm, :].T
    qb = p['q_b'][perm][None, :]
    kvt = jnp.concatenate([p['k_w'][perm, :].T, p['v_w'][perm, :].T], axis=1)
    kvb = jnp.concatenate([p['k_b'][perm], p['v_b'][perm]])[None, :]
    wm = p['m_w'][:, perm].T
    mb = p['m_b'][None, :]
    wa1 = p['mlp1_w'][:, :256].T
    wb1 = p['mlp1_w'][:, 256:].T
    b1 = p['mlp1_b'][None, :]
    w2t = p['mlp2_w'].T
    b2 = p['mlp2_b'][None, :]
    return qt, qb, kvt, kvb, wa1, wb1, wm, mb, b1, w2t, b2


def _self_att(X, idx_flat, cnt, wt1, wt2, wa, wb, wc):
    a1, b1 = _lin1(X, wt1)
    s1, m1 = _gather_reduce(b1, idx_flat)
    x1, a2, b2 = _stage2(a1, b1, s1, m1, cnt, wt2)
    s2, m2 = _gather_reduce(b2, idx_flat)
    return _stage3(X, x1, a2, b2, s2, m2, cnt, wa, wb, wc)


def _att_prop(S, T, ap):
    qt, qb, kvt, kvb, wa1, wb1, wm, mb, b1, w2t, b2 = ap
    qp = _linear(S, qt, qb)
    kvp = _linear(T, kvt, kvb)
    ao = _mha(qp, kvp)
    h1 = _mlp1(S, ao, wa1, wb1, wm, mb, b1)
    return _mlp2(h1, w2t, b2, S)


@jax.jit
def _impl(src_points, src_feats, tgt_points, tgt_feats, params):
    X0 = jnp.concatenate([src_feats[0].T, tgt_feats[0].T], axis=0)
    pts = jnp.concatenate([src_points[0].T, tgt_points[0].T], axis=0)
    pts8 = jnp.pad(pts, ((0, 0), (0, 5)))
    idx, cnt = _knn(pts8)
    idx_flat = idx.reshape(-1)

    sa1 = _prep_sa(params['sa1'])
    sa2 = _prep_sa(params['sa2'])
    ap = _prep_ap(params['ap'])

    Xa = _self_att(X0, idx_flat, cnt, *sa1)
    sf, tf = Xa[:N], Xa[N:]
    sf = _att_prop(sf, tf, ap)
    tf = _att_prop(tf, sf, ap)
    Xb = jnp.concatenate([sf, tf], axis=0)
    Xc = _self_att(Xb, idx_flat, cnt, *sa2)

    fc, sc = _proj(Xc, params['proj_gnn_w'].T, params['proj_gnn_b'][None, :],
                   params['proj_score_w'].T, params['proj_score_b'][None, :])
    ss, ts = sc[:N], sc[N:]
    eps = jnp.reshape(params['epsilon'], (1, 1))
    s1, s2 = _final(fc, ss, ts, eps)
    sal = jnp.concatenate([s1, s2], axis=0)
    return jnp.concatenate([sc, sal, fc], axis=1)


def kernel(src_points, src_feats, tgt_points, tgt_feats, params):
    return _impl(src_points, src_feats, tgt_points, tgt_feats, params)


# speedup vs baseline: 6.2036x; 1.0247x over previous
import functools

import jax
import jax.numpy as jnp
from jax import lax
from jax.experimental import pallas as pl
from jax.experimental.pallas import tpu as pltpu
from jax.experimental.pallas import tpu_sc as plsc

F32 = jnp.float32
N = 2048
NS = 4096
KNN = 10
NCORES = 2
NSUB = 16
NW = NCORES * NSUB
QPW = NS // NW
QC = 8
IPC = QC * KNN
NCHUNK = QPW // QC


def _leaky(x):
    return jnp.where(x >= 0, x, 0.2 * x)


_HI = lax.Precision.HIGHEST


def _dot(a, b, precision=None):
    return jnp.dot(a, b, preferred_element_type=F32, precision=precision)


def _hdot(a, b):
    return jnp.dot(a, b, preferred_element_type=F32, precision=_HI)


def _cdot(a, b, precision=None):
    return lax.dot_general(a, b, (((1,), (1,)), ((), ())),
                           preferred_element_type=F32, precision=precision)


def _tdot(a, b):
    return lax.dot_general(a, b, (((0,), (0,)), ((), ())),
                           preferred_element_type=F32)


def _knn_body(pq_ref, pa_ref, idx_ref, cnt_ref):
    b = pl.program_id(0)
    pq = pq_ref[...]
    pa = pa_ref[...]
    g = _cdot(pq, pa)
    sqq = jnp.sum(pq * pq, axis=1, keepdims=True)
    sqa = jnp.transpose(jnp.sum(pa * pa, axis=1, keepdims=True))
    d = ((-2.0 * g) + sqq) + sqa
    d = jnp.maximum(d, 1e-12)
    colid = lax.broadcasted_iota(jnp.int32, (256, 2048), 1)
    big = jnp.float32(3.0e38)
    outs = []
    cnt = jnp.zeros((1, 2048), F32)
    for j in range(KNN + 1):
        rmin = jnp.min(d, axis=1, keepdims=True)
        amin = jnp.min(jnp.where(d <= rmin, colid, 2048), axis=1, keepdims=True)
        outs.append(amin)
        onehot = colid == amin
        if j > 0:
            cnt = cnt + jnp.sum(onehot.astype(F32), axis=0, keepdims=True)
        d = jnp.where(onehot, big, d)
    idx_ref[...] = jnp.concatenate(outs[1:], axis=1) + (b // 8) * 2048

    @pl.when(b % 8 == 0)
    def _():
        cnt_ref[...] = jnp.zeros((1, 2048), F32)
    cnt_ref[...] += cnt


def _knn(pts8):
    return pl.pallas_call(
        _knn_body,
        grid=(16,),
        in_specs=[
            pl.BlockSpec((256, 8), lambda b: (b, 0)),
            pl.BlockSpec((2048, 8), lambda b: (b // 8, 0)),
        ],
        out_specs=[pl.BlockSpec((256, KNN), lambda b: (b, 0)),
                   pl.BlockSpec((1, 2048), lambda b: (0, b // 8))],
        out_shape=[jax.ShapeDtypeStruct((NS, KNN), jnp.int32),
                   jax.ShapeDtypeStruct((1, NS), F32)],
    )(pts8, pts8)


@functools.lru_cache(maxsize=None)
def _make_gather_reduce(C):
    mesh = plsc.VectorSubcoreMesh(core_axis_name="c", subcore_axis_name="s")

    @functools.partial(
        pl.kernel,
        mesh=mesh,
        out_type=[jax.ShapeDtypeStruct((NS, C), F32)] * 2,
        scratch_types=[
            pltpu.VMEM((QPW * KNN,), jnp.int32),
            pltpu.VMEM((IPC, C), F32),
            pltpu.VMEM((IPC, C), F32),
            pltpu.VMEM((QC, C), F32),
            pltpu.VMEM((QC, C), F32),
            pltpu.VMEM((QC, C), F32),
            pltpu.VMEM((QC, C), F32),
            pltpu.SemaphoreType.DMA,
            pltpu.SemaphoreType.DMA,
            pltpu.SemaphoreType.DMA,
            pltpu.SemaphoreType.DMA,
        ],
    )
    def gr(table_hbm, idx_hbm, s_hbm, m_hbm,
           idx_v, rows_a, rows_b, s_a, m_a, s_b, m_b,
           sem_a, sem_b, osem_a, osem_b):
        wid = lax.axis_index("s") * NCORES + lax.axis_index("c")
        qbase = wid * QPW
        pltpu.sync_copy(idx_hbm.at[pl.ds(qbase * KNN, QPW * KNN)], idx_v)

        def start(c, rows, sem):
            pltpu.async_copy(table_hbm.at[idx_v.at[pl.ds(c * IPC, IPC)]],
                             rows, sem)

        def wait(rows, sem):
            pltpu.make_async_copy(
                table_hbm.at[idx_v.at[pl.ds(0, IPC)]], rows, sem).wait()

        def compute(c, rows, sv, mv, osem):
            def qloop(qi, carry):
                for j in range(C // 16):
                    sl = pl.ds(j * 16, 16)
                    r = rows[qi * KNN, sl]
                    acc_s = r
                    acc_m = r
                    for k in range(1, KNN):
                        r = rows[qi * KNN + k, sl]
                        acc_s = acc_s + r
                        acc_m = jnp.maximum(acc_m, r)
                    sv[qi, sl] = acc_s
                    mv[qi, sl] = acc_m
                return carry
            lax.fori_loop(0, QC, qloop, 0)
            off = qbase + c * QC
            pltpu.async_copy(sv, s_hbm.at[pl.ds(off, QC)], osem)
            pltpu.async_copy(mv, m_hbm.at[pl.ds(off, QC)], osem)

        def drain(sv, mv, osem):
            pltpu.make_async_copy(sv, s_hbm.at[pl.ds(0, QC)], osem).wait()
            pltpu.make_async_copy(mv, m_hbm.at[pl.ds(0, QC)], osem).wait()

        start(0, rows_a, sem_a)

        def body(g2, carry):
            c0 = g2 * 2
            start(c0 + 1, rows_b, sem_b)
            wait(rows_a, sem_a)

            @pl.when(g2 > 0)
            def _():
                drain(s_a, m_a, osem_a)

            compute(c0, rows_a, s_a, m_a, osem_a)

            @pl.when(g2 < NCHUNK // 2 - 1)
            def _():
                start(c0 + 2, rows_a, sem_a)

            wait(rows_b, sem_b)

            @pl.when(g2 > 0)
            def _():
                drain(s_b, m_b, osem_b)

            compute(c0 + 1, rows_b, s_b, m_b, osem_b)
            return carry

        lax.fori_loop(0, NCHUNK // 2, body, 0)
        drain(s_a, m_a, osem_a)
        drain(s_b, m_b, osem_b)

    return gr


def _gather_reduce(table, idx_flat):
    return _make_gather_reduce(table.shape[1])(table, idx_flat)


def _lin1_body(x_ref, w_ref, a_ref, b_ref):
    y = _dot(x_ref[...], w_ref[...])
    a_ref[...] = y[:, :256]
    b_ref[...] = y[:, 256:]


def _lin1(x, wt):
    return pl.pallas_call(
        _lin1_body,
        out_shape=[jax.ShapeDtypeStruct((NS, 256), F32)] * 2,
    )(x, wt)


def _edge_stats(ap, b, s, mx, cnt):
    a = ap - b
    nk = jnp.float32(N * KNN)
    kf = jnp.float32(KNN)
    cw = jnp.transpose(cnt)
    mean = (jnp.sum(a, axis=0, keepdims=True) * kf
            + jnp.sum(s, axis=0, keepdims=True)) / nk
    msq = (jnp.sum(a * a, axis=0, keepdims=True) * kf
           + 2.0 * jnp.sum(a * s, axis=0, keepdims=True)
           + jnp.sum(b * b * cw, axis=0, keepdims=True)) / nk
    var = msq - mean * mean
    r = lax.rsqrt(var + 1e-5)
    return _leaky((a + mx - mean) * r)


def _stage2_body(a_ref, b_ref, s_ref, m_ref, cnt_ref, w_ref,
                 x1_ref, a2_ref, b2_ref):
    x1 = _edge_stats(a_ref[...], b_ref[...], s_ref[...], m_ref[...],
                     cnt_ref[...])
    x1_ref[...] = x1
    y = _dot(x1, w_ref[...])
    a2_ref[...] = y[:, :512]
    b2_ref[...] = y[:, 512:]


def _stage2(a1, b1, s1, m1, cnt, wt2):
    spec = pl.BlockSpec((N, 256), lambda h: (h, 0))
    return pl.pallas_call(
        _stage2_body,
        grid=(2,),
        in_specs=[spec, spec, spec, spec,
                  pl.BlockSpec((1, N), lambda h: (0, h)),
                  pl.BlockSpec((256, 1024), lambda h: (0, 0))],
        out_specs=[pl.BlockSpec((N, 256), lambda h: (h, 0)),
                   pl.BlockSpec((N, 512), lambda h: (h, 0)),
                   pl.BlockSpec((N, 512), lambda h: (h, 0))],
        out_shape=[jax.ShapeDtypeStruct((NS, 256), F32),
                   jax.ShapeDtypeStruct((NS, 512), F32),
                   jax.ShapeDtypeStruct((NS, 512), F32)],
    )(a1, b1, s1, m1, cnt, wt2)


def _stage3a_body(a_ref, b_ref, s_ref, m_ref, cnt_ref, x2_ref):
    x2_ref[...] = _edge_stats(a_ref[...], b_ref[...], s_ref[...], m_ref[...],
                              cnt_ref[...])


def _stage3b_body(x0_ref, x1_ref, x2_ref, wa_ref, wb_ref, wc_ref, x3_ref):
    y = (_dot(x0_ref[...], wa_ref[...])
         + _dot(x1_ref[...], wb_ref[...])
         + _dot(x2_ref[...], wc_ref[...]))
    mu = jnp.mean(y, axis=0, keepdims=True)
    d = y - mu
    v = jnp.mean(d * d, axis=0, keepdims=True)
    x3_ref[...] = _leaky(d * lax.rsqrt(v + 1e-5))


def _stage3(x0, x1, a2, b2, s2, m2, cnt, wa, wb, wc):
    spec256 = pl.BlockSpec((N, 256), lambda h: (h, 0))
    spec512 = pl.BlockSpec((N, 512), lambda h: (h, 0))
    wspec = lambda shape: pl.BlockSpec(shape, lambda h: (0, 0))
    x2 = pl.pallas_call(
        _stage3a_body,
        grid=(2,),
        in_specs=[spec512, spec512, spec512, spec512,
                  pl.BlockSpec((1, N), lambda h: (0, h))],
        out_specs=spec512,
        out_shape=jax.ShapeDtypeStruct((NS, 512), F32),
    )(a2, b2, s2, m2, cnt)
    return pl.pallas_call(
        _stage3b_body,
        grid=(2,),
        in_specs=[spec256, spec256, spec512,
                  wspec((256, 256)), wspec((256, 256)), wspec((512, 256))],
        out_specs=spec256,
        out_shape=jax.ShapeDtypeStruct((NS, 256), F32),
    )(x0, x1, x2, wa, wb, wc)


def _linear_body(x_ref, w_ref, b_ref, o_ref):
    o_ref[...] = (_dot(x_ref[...], w_ref[...])
                  + b_ref[...])


def _linear(x, wt, b):
    m, cout = x.shape[0], wt.shape[1]
    return pl.pallas_call(
        _linear_body,
        out_shape=jax.ShapeDtypeStruct((m, cout), F32),
    )(x, wt, b)


def _mha_body(q_ref, k_ref, v_ref, o_ref):
    s = _cdot(q_ref[0], k_ref[0]) * 0.125
    m = jnp.max(s, axis=1, keepdims=True)
    e = jnp.exp(s - m)
    den = jnp.sum(e, axis=1, keepdims=True)
    o_ref[0] = _dot(e, v_ref[0]) / den


def _mha(qp, kvp):
    q3 = qp.reshape(N, 4, 64).transpose(1, 0, 2)
    k3 = kvp[:, :256].reshape(N, 4, 64).transpose(1, 0, 2)
    v3 = kvp[:, 256:].reshape(N, 4, 64).transpose(1, 0, 2)
    hspec = pl.BlockSpec((1, N, 64), lambda h: (h, 0, 0))
    o3 = pl.pallas_call(
        _mha_body,
        grid=(4,),
        in_specs=[hspec, hspec, hspec],
        out_specs=hspec,
        out_shape=jax.ShapeDtypeStruct((4, N, 64), F32),
    )(q3, k3, v3)
    return o3.transpose(1, 0, 2).reshape(N, 256)


def _mlp1_body(s_ref, a_ref, wa_ref, wb_ref, wm_ref, mb_ref, b_ref, o_ref):
    msg = _dot(a_ref[...], wm_ref[...]) + mb_ref[...]
    y = (_dot(s_ref[...], wa_ref[...])
         + _dot(msg, wb_ref[...])
         + b_ref[...])
    mu = jnp.mean(y, axis=0, keepdims=True)
    d = y - mu
    v = jnp.mean(d * d, axis=0, keepdims=True)
    o_ref[...] = jnp.maximum(d * lax.rsqrt(v + 1e-5), 0.0)


def _mlp1(s, a, wa, wb, wm, mb, b):
    return pl.pallas_call(
        _mlp1_body,
        out_shape=jax.ShapeDtypeStruct((N, 512), F32),
    )(s, a, wa, wb, wm, mb, b)


def _mlp2_body(h_ref, w_ref, b_ref, res_ref, o_ref):
    o_ref[...] = (_dot(h_ref[...], w_ref[...])
                  + b_ref[...] + res_ref[...])


def _mlp2(h, wt, b, res):
    return pl.pallas_call(
        _mlp2_body,
        out_shape=jax.ShapeDtypeStruct((N, 256), F32),
    )(h, wt, b, res)


def _proj_body(x_ref, wg_ref, bg_ref, ws_ref, bs_ref, fc_ref, sc_ref):
    fc = (jnp.dot(x_ref[...], wg_ref[...], preferred_element_type=F32)
          + bg_ref[...])
    fc_ref[...] = fc
    sc_ref[...] = (_dot(fc, ws_ref[...])
                   + bs_ref[...])


def _proj(x, wg, bg, ws, bs):
    return pl.pallas_call(
        _proj_body,
        out_shape=[jax.ShapeDtypeStruct((NS, 256), F32),
                   jax.ShapeDtypeStruct((NS, 1), F32)],
    )(x, wg, bg, ws, bs)


def _norm_inner(fc_ref, eps_ref):
    fg = fc_ref[...]
    nr = jnp.maximum(jnp.sqrt(jnp.sum(fg * fg, axis=1, keepdims=True)), 1e-12)
    g = fg / nr
    temp = jnp.exp(eps_ref[0, 0]) + 0.03
    return _cdot(g[:N], g[N:]) / temp


def _final1_body(fc_ref, ts_ref, eps_ref, s1_ref):
    inner = _norm_inner(fc_ref, eps_ref)
    m1 = jnp.max(inner, axis=1, keepdims=True)
    e1 = jnp.exp(inner - m1)
    s1_ref[...] = (_dot(e1, ts_ref[...])
                   / jnp.sum(e1, axis=1, keepdims=True))


def _final2_body(fc_ref, ss_ref, eps_ref, s2_ref):
    inner = _norm_inner(fc_ref, eps_ref)
    m2 = jnp.max(inner, axis=0, keepdims=True)
    e2 = jnp.exp(inner - m2)
    p2 = e2 / jnp.sum(e2, axis=0, keepdims=True)
    s2_ref[...] = _tdot(p2, ss_ref[...])


def _final(fc, ss, ts, eps):
    s1 = pl.pallas_call(
        _final1_body,
        out_shape=jax.ShapeDtypeStruct((N, 1), F32),
    )(fc, ts, eps)
    s2 = pl.pallas_call(
        _final2_body,
        out_shape=jax.ShapeDtypeStruct((N, 1), F32),
    )(fc, ss, eps)
    return s1, s2


def _prep_sa(p):
    c1 = p['conv1']
    wt1 = jnp.concatenate([c1[:, :256].T, c1[:, 256:].T], axis=1)
    c2 = p['conv2']
    wt2 = jnp.concatenate([c2[:, :256].T, c2[:, 256:].T], axis=1)
    c3 = p['conv3']
    return wt1, wt2, c3[:, :256].T, c3[:, 256:512].T, c3[:, 512:].T


def _prep_ap(p):
    perm = jnp.arange(256).reshape(64, 4).T.reshape(-1)
    qt = p['q_w'][perm, :].T
    qb = p['q_b'][perm][None, :]
    kvt = jnp.concatenate([p['k_w'][perm, :].T, p['v_w'][perm, :].T], axis=1)
    kvb = jnp.concatenate([p['k_b'][perm], p['v_b'][perm]])[None, :]
    wm = p['m_w'][:, perm].T
    mb = p['m_b'][None, :]
    wa1 = p['mlp1_w'][:, :256].T
    wb1 = p['mlp1_w'][:, 256:].T
    b1 = p['mlp1_b'][None, :]
    w2t = p['mlp2_w'].T
    b2 = p['mlp2_b'][None, :]
    return qt, qb, kvt, kvb, wa1, wb1, wm, mb, b1, w2t, b2


def _self_att(X, idx_flat, cnt, wt1, wt2, wa, wb, wc):
    a1, b1 = _lin1(X, wt1)
    s1, m1 = _gather_reduce(b1, idx_flat)
    x1, a2, b2 = _stage2(a1, b1, s1, m1, cnt, wt2)
    s2, m2 = _gather_reduce(b2, idx_flat)
    return _stage3(X, x1, a2, b2, s2, m2, cnt, wa, wb, wc)


def _att_prop(S, T, ap):
    qt, qb, kvt, kvb, wa1, wb1, wm, mb, b1, w2t, b2 = ap
    qp = _linear(S, qt, qb)
    kvp = _linear(T, kvt, kvb)
    ao = _mha(qp, kvp)
    h1 = _mlp1(S, ao, wa1, wb1, wm, mb, b1)
    return _mlp2(h1, w2t, b2, S)


@jax.jit
def _impl(src_points, src_feats, tgt_points, tgt_feats, params):
    X0 = jnp.concatenate([src_feats[0].T, tgt_feats[0].T], axis=0)
    pts = jnp.concatenate([src_points[0].T, tgt_points[0].T], axis=0)
    pts8 = jnp.pad(pts, ((0, 0), (0, 5)))
    idx, cnt = _knn(pts8)
    idx_flat = idx.reshape(-1)

    sa1 = _prep_sa(params['sa1'])
    sa2 = _prep_sa(params['sa2'])
    ap = _prep_ap(params['ap'])

    Xa = _self_att(X0, idx_flat, cnt, *sa1)
    sf, tf = Xa[:N], Xa[N:]
    sf = _att_prop(sf, tf, ap)
    tf = _att_prop(tf, sf, ap)
    Xb = jnp.concatenate([sf, tf], axis=0)
    Xc = _self_att(Xb, idx_flat, cnt, *sa2)

    fc, sc = _proj(Xc, params['proj_gnn_w'].T, params['proj_gnn_b'][None, :],
                   params['proj_score_w'].T, params['proj_score_b'][None, :])
    ss, ts = sc[:N], sc[N:]
    eps = jnp.reshape(params['epsilon'], (1, 1))
    s1, s2 = _final(fc, ss, ts, eps)
    sal = jnp.concatenate([s1, s2], axis=0)
    return jnp.concatenate([sc, sal, fc], axis=1)


def kernel(src_points, src_feats, tgt_points, tgt_feats, params):
    return _impl(src_points, src_feats, tgt_points, tgt_feats, params)
